# baseline (speedup 1.0000x reference)
import functools

import jax
import jax.numpy as jnp
from jax import lax
from jax.experimental import pallas as pl
from jax.experimental.pallas import tpu as pltpu
from jax.experimental.pallas import tpu_sc as plsc

B = 4096
NC, NS = 2, 16
NW = NC * NS
EPW = B // NW
BB = 4
NIT = EPW // BB
KIN = 55
KOUT = 75
KINP = 72
KOUTP = 96


def _sc_dots(t_in, t_out, iin_ln, iout_ln, iin_off, iout_off):
    mesh = plsc.VectorSubcoreMesh(core_axis_name="c", subcore_axis_name="s")

    @functools.partial(
        pl.kernel,
        out_type=jax.ShapeDtypeStruct((B, 32), jnp.float32),
        mesh=mesh,
        scratch_types=[
            pltpu.VMEM((8, KIN), jnp.int32),
            pltpu.VMEM((8, KOUT), jnp.int32),
            pltpu.VMEM((8, KINP), jnp.int32),
            pltpu.VMEM((8, KOUTP), jnp.int32),
            pltpu.VMEM((BB, KIN, 128), jnp.float32),
            pltpu.VMEM((BB, KOUT, 128), jnp.float32),
            pltpu.VMEM((8, 32), jnp.float32),
            pltpu.SemaphoreType.DMA,
        ],
        compiler_params=pltpu.CompilerParams(use_tc_tiling_on_sc=False),
    )
    def k(t_in_h, t_out_h, iinl_h, ioutl_h, iino_h, iouto_h,
          out_h, iinl_v, ioutl_v, iino_v, iouto_v, lI, lO, dv, sem):
        wid = lax.axis_index("c") * NS + lax.axis_index("s")
        lane = lax.iota(jnp.int32, 16)

        def hsum(v):
            for kk in (8, 4, 2, 1):
                v = v + v.at[jnp.bitwise_xor(lane, kk)].get(
                    mode="promise_in_bounds")
            return v

        def it_body(it, _):
            row8 = wid * EPW + (it // 2) * 8

            @pl.when(it % 2 == 0)
            def _stage():
                pltpu.sync_copy(iinl_h.at[pl.ds(row8, 8)], iinl_v)
                pltpu.sync_copy(ioutl_h.at[pl.ds(row8, 8)], ioutl_v)
                pltpu.sync_copy(iino_h.at[pl.ds(row8, 8)], iino_v)
                pltpu.sync_copy(iouto_h.at[pl.ds(row8, 8)], iouto_v)

            cps = []
            for e in range(BB):
                i = (it % 2) * BB + e
                cps.append(pltpu.async_copy(
                    t_in_h.at[iinl_v.at[i]], lI.at[e], sem))
                cps.append(pltpu.async_copy(
                    t_out_h.at[ioutl_v.at[i]], lO.at[e], sem))
            for c in cps:
                c.wait()

            for e in range(BB):
                i = (it % 2) * BB + e

                def row_off(offs_v, r):
                    chunk = offs_v[i, pl.ds(r, 16)]
                    return chunk[0]

                def load_row(lb, offs_v, r):
                    o = row_off(offs_v, r)
                    return (
                        lb[e, r, pl.ds(o, 16)],
                        lb[e, r, pl.ds(o + 16, 16)],
                        lb[e, r, pl.ds(o + 32, 16)],
                        lb[e, r, pl.ds(o + 48, 16)],
                    )

                ial, iah, iml, imh = load_row(lI, iino_v, 0)
                for t in range(1, 5):
                    al, ah, ml, mh = load_row(lI, iino_v, t)
                    ial = ial + al
                    iah = iah + ah
                    iml = iml * ml
                    imh = imh * mh

                def dot_group(lb, offs_v, gb):
                    gal, gah, gml, gmh = load_row(lb, offs_v, gb)
                    for t in range(1, 5):
                        al, ah, ml, mh = load_row(lb, offs_v, gb + t)
                        gal = gal + al
                        gah = gah + ah
                        gml = gml * ml
                        gmh = gmh * mh
                    v = ial * gal + iah * gah + iml * gml + imh * gmh
                    return hsum(v)

                def g_sa(g, acc):
                    d = dot_group(lI, iino_v, 5 + g * 5)
                    return jnp.where(lane == g, d, acc)

                acc1 = lax.fori_loop(0, 10, g_sa, jnp.zeros(16, jnp.float32))

                def g_pn(g, acc):
                    d = dot_group(lO, iouto_v, g * 5)
                    return jnp.where(lane == g, d, acc)

                acc0 = lax.fori_loop(0, 15, g_pn, jnp.zeros(16, jnp.float32))

                dv[i, pl.ds(0, 16)] = acc0
                dv[i, pl.ds(16, 16)] = acc1

            @pl.when(it % 2 == 1)
            def _flush():
                pltpu.sync_copy(dv, out_h.at[pl.ds(row8, 8)])

            return 0

        lax.fori_loop(0, NIT, it_body, 0)

    return k(t_in, t_out, iin_ln, iout_ln, iin_off, iout_off)


_TCB = 16384


def _tc_pack2_body(xa_ref, xb_ref, o_ref):
    xa = xa_ref[...]
    xb = xb_ref[...]
    i32t = lax.broadcasted_iota(jnp.int32, (32, 32), 0)
    eye = (i32t == lax.broadcasted_iota(jnp.int32, (32, 32), 1)).astype(
        jnp.float32)
    dn = (((0,), (0,)), ((), ()))

    def tp_xlu(x, lo):
        return lax.transpose(x[:, lo:lo + 512], (1, 0))

    def tp_mxu(x, lo):
        return lax.dot_general(x[:, lo:lo + 512], eye, dn,
                               preferred_element_type=jnp.float32)

    rows = []
    for j in range(16):
        c0 = j * 1024
        tp = tp_xlu if j % 2 == 0 else tp_mxu
        rows.append(jnp.concatenate(
            [tp(xa, c0), tp(xb, c0), tp(xa, c0 + 512), tp(xb, c0 + 512)],
            axis=1))
    o_ref[...] = jnp.concatenate(rows, axis=0)


def _tc_pack2(ta, tb):
    eta = ta.T
    etb = tb.T
    nblk = (eta.shape[1] + _TCB - 1) // _TCB
    return pl.pallas_call(
        _tc_pack2_body,
        grid=(nblk,),
        in_specs=[pl.BlockSpec((32, _TCB), lambda g: (0, g)),
                  pl.BlockSpec((32, _TCB), lambda g: (0, g))],
        out_specs=pl.BlockSpec((8192, 128), lambda g: (g, 0)),
        out_shape=jax.ShapeDtypeStruct((500000, 128), jnp.float32),
    )(eta, etb)


def _tc_body(dots_ref, ms_ref, ma_ref, out_ref):
    eps = 1e-10
    x = dots_ref[...]
    col = lax.broadcasted_iota(jnp.int32, x.shape, 1)

    def softplus(z):
        return jnp.maximum(z, 0.0) + jnp.log1p(jnp.exp(-jnp.abs(z)))

    pos = softplus(-x - eps)
    neg = softplus(x - eps)
    ms = ms_ref[...]
    ma = ma_ref[...]
    zero = jnp.zeros_like(x)
    p_s = jnp.sum(jnp.where(col < 5, pos, zero))
    n_s = jnp.sum(jnp.where((col >= 5) & (col < 15), neg, zero))
    s_s = jnp.sum(jnp.where((col >= 16) & (col < 21), ms * pos, zero))
    a_s = jnp.sum(jnp.where((col >= 21) & (col < 26), ma * neg, zero))
    lane = lax.broadcasted_iota(jnp.int32, (1, 128), 1)
    out_ref[...] = jnp.where(
        lane == 0, p_s,
        jnp.where(lane == 1, n_s,
                  jnp.where(lane == 2, s_s,
                            jnp.where(lane == 3, a_s, 0.0))))


def _tc_loss(dots, ms_ix, ma_ix):
    return pl.pallas_call(
        _tc_body,
        out_shape=jax.ShapeDtypeStruct((1, 128), jnp.float32),
    )(dots, ms_ix, ma_ix)


def kernel(w_ix, p_ix, n_ix, s_ix, ms_ix, a_ix, ma_ix, e_ia, e_im, e_oa, e_om):
    t_in = _tc_pack2(e_ia, e_im)
    t_out = _tc_pack2(e_oa, e_om)

    w = w_ix.reshape(B, 5).astype(jnp.int32)
    p = p_ix.reshape(B, 25).astype(jnp.int32)
    n = n_ix.reshape(B, 50).astype(jnp.int32)
    s = s_ix.reshape(B, 25).astype(jnp.int32)
    a = a_ix.reshape(B, 25).astype(jnp.int32)
    iin_raw = jnp.concatenate([w, s, a], axis=1)
    iout_raw = jnp.concatenate([p, n], axis=1)

    def to_line(ix):
        return ((ix >> 10) << 9) | (ix & 511)

    def to_off(ix):
        return ((ix >> 9) & 1) * 64

    iin_ln = to_line(iin_raw)
    iout_ln = to_line(iout_raw)
    iin_off = jnp.pad(to_off(iin_raw), ((0, 0), (0, KINP - KIN)))
    iout_off = jnp.pad(to_off(iout_raw), ((0, 0), (0, KOUTP - KOUT)))

    dots = _sc_dots(t_in, t_out, iin_ln, iout_ln, iin_off, iout_off)
    out = _tc_loss(dots, ms_ix.astype(jnp.float32), ma_ix.astype(jnp.float32))
    p_s = out[0, 0] / B
    n_s = out[0, 1] / B
    s_s = out[0, 2] / B
    a_s = out[0, 3] / B
    loss = p_s + n_s + s_s + a_s
    return (loss, p_s, n_s, s_s, a_s)


# speedup vs baseline: 1.0061x; 1.0061x over previous
import functools

import jax
import jax.numpy as jnp
from jax import lax
from jax.experimental import pallas as pl
from jax.experimental.pallas import tpu as pltpu
from jax.experimental.pallas import tpu_sc as plsc

B = 4096
NC, NS = 2, 16
NW = NC * NS
EPW = B // NW
BB = 4
NIT = EPW // BB
KIN = 55
KOUT = 75
KINP = 72
KOUTP = 96


def _sc_dots(t_in, t_out, iin_ln, iout_ln, iin_off, iout_off):
    mesh = plsc.VectorSubcoreMesh(core_axis_name="c", subcore_axis_name="s")

    @functools.partial(
        pl.kernel,
        out_type=jax.ShapeDtypeStruct((B, 32), jnp.float32),
        mesh=mesh,
        scratch_types=[
            pltpu.VMEM((8, KIN), jnp.int32),
            pltpu.VMEM((8, KOUT), jnp.int32),
            pltpu.VMEM((8, KINP), jnp.int32),
            pltpu.VMEM((8, KOUTP), jnp.int32),
            pltpu.VMEM((BB, KIN, 128), jnp.float32),
            pltpu.VMEM((BB, KOUT, 128), jnp.float32),
            pltpu.VMEM((8, 32), jnp.float32),
            pltpu.SemaphoreType.DMA,
        ],
        compiler_params=pltpu.CompilerParams(use_tc_tiling_on_sc=False),
    )
    def k(t_in_h, t_out_h, iinl_h, ioutl_h, iino_h, iouto_h,
          out_h, iinl_v, ioutl_v, iino_v, iouto_v, lI, lO, dv, sem):
        wid = lax.axis_index("c") * NS + lax.axis_index("s")
        lane = lax.iota(jnp.int32, 16)

        def hsum(v):
            for kk in (8, 4, 2, 1):
                v = v + v.at[jnp.bitwise_xor(lane, kk)].get(
                    mode="promise_in_bounds")
            return v

        def it_body(it, _):
            row8 = wid * EPW + (it // 2) * 8

            @pl.when(it % 2 == 0)
            def _stage():
                pltpu.sync_copy(iinl_h.at[pl.ds(row8, 8)], iinl_v)
                pltpu.sync_copy(ioutl_h.at[pl.ds(row8, 8)], ioutl_v)
                pltpu.sync_copy(iino_h.at[pl.ds(row8, 8)], iino_v)
                pltpu.sync_copy(iouto_h.at[pl.ds(row8, 8)], iouto_v)

            cps = []
            for e in range(BB):
                i = (it % 2) * BB + e
                cps.append(pltpu.async_copy(
                    t_in_h.at[iinl_v.at[i]], lI.at[e], sem))
                cps.append(pltpu.async_copy(
                    t_out_h.at[ioutl_v.at[i]], lO.at[e], sem))
            for c in cps:
                c.wait()

            for e in range(BB):
                i = (it % 2) * BB + e

                def row_off(offs_v, r):
                    chunk = offs_v[i, pl.ds(r, 16)]
                    return chunk[0]

                def load_row(lb, offs_v, r):
                    o = row_off(offs_v, r)
                    return (
                        lb[e, r, pl.ds(o, 16)],
                        lb[e, r, pl.ds(o + 16, 16)],
                        lb[e, r, pl.ds(o + 32, 16)],
                        lb[e, r, pl.ds(o + 48, 16)],
                    )

                ial, iah, iml, imh = load_row(lI, iino_v, 0)
                for t in range(1, 5):
                    al, ah, ml, mh = load_row(lI, iino_v, t)
                    ial = ial + al
                    iah = iah + ah
                    iml = iml * ml
                    imh = imh * mh

                def dot_group(lb, offs_v, gb):
                    gal, gah, gml, gmh = load_row(lb, offs_v, gb)
                    for t in range(1, 5):
                        al, ah, ml, mh = load_row(lb, offs_v, gb + t)
                        gal = gal + al
                        gah = gah + ah
                        gml = gml * ml
                        gmh = gmh * mh
                    v = ial * gal + iah * gah + iml * gml + imh * gmh
                    return hsum(v)

                def g_sa(g, acc):
                    d = dot_group(lI, iino_v, 5 + g * 5)
                    return jnp.where(lane == g, d, acc)

                acc1 = lax.fori_loop(0, 10, g_sa, jnp.zeros(16, jnp.float32))

                def g_pn(g, acc):
                    d = dot_group(lO, iouto_v, g * 5)
                    return jnp.where(lane == g, d, acc)

                acc0 = lax.fori_loop(0, 15, g_pn, jnp.zeros(16, jnp.float32))

                dv[i, pl.ds(0, 16)] = acc0
                dv[i, pl.ds(16, 16)] = acc1

            @pl.when(it % 2 == 1)
            def _flush():
                pltpu.sync_copy(dv, out_h.at[pl.ds(row8, 8)])

            return 0

        lax.fori_loop(0, NIT, it_body, 0)

    return k(t_in, t_out, iin_ln, iout_ln, iin_off, iout_off)


_TCB = 8192


def _tc_pack2_body(xa_ref, xb_ref, o_ref):
    xa = xa_ref[...]
    xb = xb_ref[...]
    i32t = lax.broadcasted_iota(jnp.int32, (32, 32), 0)
    eye = (i32t == lax.broadcasted_iota(jnp.int32, (32, 32), 1)).astype(
        jnp.float32)
    dn = (((0,), (0,)), ((), ()))

    def tp(x, lo):
        return lax.transpose(x[:, lo:lo + 512], (1, 0))

    rows = []
    for j in range(8):
        c0 = j * 1024
        rows.append(jnp.concatenate(
            [tp(xa, c0), tp(xb, c0), tp(xa, c0 + 512), tp(xb, c0 + 512)],
            axis=1))
    o_ref[...] = jnp.concatenate(rows, axis=0)


def _tc_pack2(ta, tb):
    eta = ta.T
    etb = tb.T
    nblk = (eta.shape[1] + _TCB - 1) // _TCB
    return pl.pallas_call(
        _tc_pack2_body,
        grid=(nblk,),
        in_specs=[pl.BlockSpec((32, _TCB), lambda g: (0, g)),
                  pl.BlockSpec((32, _TCB), lambda g: (0, g))],
        out_specs=pl.BlockSpec((4096, 128), lambda g: (g, 0)),
        out_shape=jax.ShapeDtypeStruct((500000, 128), jnp.float32),
    )(eta, etb)


def _tc_body(dots_ref, ms_ref, ma_ref, out_ref):
    eps = 1e-10
    x = dots_ref[...]
    col = lax.broadcasted_iota(jnp.int32, x.shape, 1)

    def softplus(z):
        return jnp.maximum(z, 0.0) + jnp.log1p(jnp.exp(-jnp.abs(z)))

    pos = softplus(-x - eps)
    neg = softplus(x - eps)
    ms = ms_ref[...]
    ma = ma_ref[...]
    zero = jnp.zeros_like(x)
    p_s = jnp.sum(jnp.where(col < 5, pos, zero))
    n_s = jnp.sum(jnp.where((col >= 5) & (col < 15), neg, zero))
    s_s = jnp.sum(jnp.where((col >= 16) & (col < 21), ms * pos, zero))
    a_s = jnp.sum(jnp.where((col >= 21) & (col < 26), ma * neg, zero))
    lane = lax.broadcasted_iota(jnp.int32, (1, 128), 1)
    out_ref[...] = jnp.where(
        lane == 0, p_s,
        jnp.where(lane == 1, n_s,
                  jnp.where(lane == 2, s_s,
                            jnp.where(lane == 3, a_s, 0.0))))


def _tc_loss(dots, ms_ix, ma_ix):
    return pl.pallas_call(
        _tc_body,
        out_shape=jax.ShapeDtypeStruct((1, 128), jnp.float32),
    )(dots, ms_ix, ma_ix)


def kernel(w_ix, p_ix, n_ix, s_ix, ms_ix, a_ix, ma_ix, e_ia, e_im, e_oa, e_om):
    t_in = _tc_pack2(e_ia, e_im)
    t_out = _tc_pack2(e_oa, e_om)

    w = w_ix.reshape(B, 5).astype(jnp.int32)
    p = p_ix.reshape(B, 25).astype(jnp.int32)
    n = n_ix.reshape(B, 50).astype(jnp.int32)
    s = s_ix.reshape(B, 25).astype(jnp.int32)
    a = a_ix.reshape(B, 25).astype(jnp.int32)
    iin_raw = jnp.concatenate([w, s, a], axis=1)
    iout_raw = jnp.concatenate([p, n], axis=1)

    def to_line(ix):
        return ((ix >> 10) << 9) | (ix & 511)

    def to_off(ix):
        return ((ix >> 9) & 1) * 64

    iin_ln = to_line(iin_raw)
    iout_ln = to_line(iout_raw)
    iin_off = jnp.pad(to_off(iin_raw), ((0, 0), (0, KINP - KIN)))
    iout_off = jnp.pad(to_off(iout_raw), ((0, 0), (0, KOUTP - KOUT)))

    dots = _sc_dots(t_in, t_out, iin_ln, iout_ln, iin_off, iout_off)
    out = _tc_loss(dots, ms_ix.astype(jnp.float32), ma_ix.astype(jnp.float32))
    p_s = out[0, 0] / B
    n_s = out[0, 1] / B
    s_s = out[0, 2] / B
    a_s = out[0, 3] / B
    loss = p_s + n_s + s_s + a_s
    return (loss, p_s, n_s, s_s, a_s)
